# baseline (speedup 1.0000x reference)
import functools

import jax
import jax.numpy as jnp
from jax import lax
from jax.experimental import pallas as pl
from jax.experimental.pallas import tpu as pltpu
from jax.experimental.pallas import tpu_sc as plsc


def _nodeproj_body(x_ref, wd_ref, ws_ref, b_ref, pd_ref, ps_ref):
    xv = x_ref[...]
    pd_ref[...] = (
        jnp.dot(xv, wd_ref[...], preferred_element_type=jnp.float32) + b_ref[...]
    )
    ps_ref[...] = jnp.dot(xv, ws_ref[...], preferred_element_type=jnp.float32)


def _nodeproj(x, wd, ws, bias):
    n, f = x.shape
    f2 = wd.shape[1]
    bn = 2000
    return pl.pallas_call(
        _nodeproj_body,
        grid=(n // bn,),
        in_specs=[
            pl.BlockSpec((bn, f), lambda i: (i, 0)),
            pl.BlockSpec((f, f2), lambda i: (0, 0)),
            pl.BlockSpec((f, f2), lambda i: (0, 0)),
            pl.BlockSpec((1, f2), lambda i: (0, 0)),
        ],
        out_specs=[
            pl.BlockSpec((bn, f2), lambda i: (i, 0)),
            pl.BlockSpec((bn, f2), lambda i: (i, 0)),
        ],
        out_shape=[jax.ShapeDtypeStruct((n, f2), jnp.float32)] * 2,
    )(x, wd, ws, bias)


def _edgeproj_body(ea_ref, we_ref, ep_ref):
    ep_ref[...] = jnp.dot(
        ea_ref[...], we_ref[...], preferred_element_type=jnp.float32
    )


def _edgeproj(edge_attr, we):
    e, d = edge_attr.shape
    f2 = we.shape[1]
    be = 3200
    return pl.pallas_call(
        _edgeproj_body,
        grid=(e // be,),
        in_specs=[
            pl.BlockSpec((be, d), lambda i: (i, 0)),
            pl.BlockSpec((d, f2), lambda i: (0, 0)),
        ],
        out_specs=pl.BlockSpec((be, f2), lambda i: (i, 0)),
        out_shape=jax.ShapeDtypeStruct((e, f2), jnp.float32),
    )(edge_attr, we)


_C = 40
_STG = 32


def _sc_body(
    dst_hbm, src_hbm, pd_hbm, ps_hbm, ep_hbm,
    agg_out,
    dsti, srci, pdv, psv, epv, msgv, stg, aggsh,
    sem0, sem1, sem2,
):
    npad = aggsh.shape[0]
    e = dst_hbm.shape[0]
    cid = lax.axis_index("c")
    sid = lax.axis_index("s")
    epc = e // 2
    epw = epc // 16
    nchunk = epw // _C
    rpt = npad // 16
    nstg = rpt // _STG

    zero16 = jnp.zeros((16,), jnp.float32)

    def _zrow(r, carry):
        for j in range(8):
            stg[r, pl.ds(16 * j, 16)] = zero16
        return carry

    lax.fori_loop(0, _STG, _zrow, 0)

    def _zcp(k, carry):
        off = sid * rpt + k * _STG
        pltpu.sync_copy(stg, aggsh.at[pl.ds(off, _STG)])
        return carry

    lax.fori_loop(0, nstg, _zcp, 0)

    plsc.subcore_barrier()

    ebase = cid * epc + sid * epw

    def _chunk(ci, carry):
        eb = ebase + ci * _C
        pltpu.sync_copy(dst_hbm.at[pl.ds(eb, _C)], dsti)
        pltpu.sync_copy(src_hbm.at[pl.ds(eb, _C)], srci)
        cp0 = pltpu.async_copy(pd_hbm.at[dsti], pdv, sem0)
        cp1 = pltpu.async_copy(ps_hbm.at[srci], psv, sem1)
        cp2 = pltpu.async_copy(ep_hbm.at[pl.ds(eb, _C)], epv, sem2)
        cp0.wait()
        cp1.wait()
        cp2.wait()

        def _row(r, rc):
            for j in range(8):
                gsl = pl.ds(16 * j, 16)
                csl = pl.ds(128 + 16 * j, 16)
                g = pdv[r, gsl] + psv[r, gsl] + epv[r, gsl]
                cz = pdv[r, csl] + psv[r, csl] + epv[r, csl]
                sg = 1.0 / (1.0 + jnp.exp(-g))
                u = jnp.exp(-jnp.abs(cz))
                t = u / (u + 2.0)
                t2 = t * t
                poly = 1.0 + t2 * (0.33333334 + t2 * (0.2 + t2 * 0.14285715))
                sp = jnp.maximum(cz, 0.0) + 2.0 * t * poly
                msgv[r, gsl] = sg * sp
            return rc

        lax.fori_loop(0, _C, _row, 0)
        pltpu.sync_copy(msgv, aggsh.at[dsti], add=True)
        return carry

    lax.fori_loop(0, nchunk, _chunk, 0)

    plsc.subcore_barrier()

    def _rd(k, carry):
        off = sid * rpt + k * _STG
        pltpu.sync_copy(aggsh.at[pl.ds(off, _STG)], stg)
        pltpu.sync_copy(stg, agg_out.at[cid, pl.ds(off, _STG)])
        return carry

    lax.fori_loop(0, nstg, _rd, 0)


def _sc_edge_pass(dst, src, pd, ps, ep, npad):
    mesh = plsc.VectorSubcoreMesh(core_axis_name="c", subcore_axis_name="s")
    kern = pl.kernel(
        _sc_body,
        mesh=mesh,
        out_type=jax.ShapeDtypeStruct((2, npad, 128), jnp.float32),
        scratch_types=[
            pltpu.VMEM((_C,), jnp.int32),
            pltpu.VMEM((_C,), jnp.int32),
            pltpu.VMEM((_C, 256), jnp.float32),
            pltpu.VMEM((_C, 256), jnp.float32),
            pltpu.VMEM((_C, 256), jnp.float32),
            pltpu.VMEM((_C, 128), jnp.float32),
            pltpu.VMEM((_STG, 128), jnp.float32),
            pltpu.VMEM_SHARED((npad, 128), jnp.float32),
            pltpu.SemaphoreType.DMA,
            pltpu.SemaphoreType.DMA,
            pltpu.SemaphoreType.DMA,
        ],
    )
    return kern(dst, src, pd, ps, ep)


_CC = 80


def _sc_count_body(dst_hbm, cnt_out, dsti, onesv, stg16, cntsh):
    npad = cntsh.shape[0]
    e = dst_hbm.shape[0]
    cid = lax.axis_index("c")
    sid = lax.axis_index("s")
    epc = e // 2
    epw = epc // 16
    nchunk = epw // _CC
    rpt = npad // 16
    nstg = rpt // _STG

    zero16 = jnp.zeros((16,), jnp.float32)
    lanes = lax.iota(jnp.int32, 16)
    onerow = jnp.where(lanes == 0, 1.0, 0.0).astype(jnp.float32)

    def _init(r, carry):
        for j in range(8):
            stg16[r, pl.ds(16 * j, 16)] = zero16
        return carry

    lax.fori_loop(0, _STG, _init, 0)

    def _orow(r, carry):
        onesv[r, pl.ds(0, 16)] = onerow
        for j in range(1, 8):
            onesv[r, pl.ds(16 * j, 16)] = zero16
        return carry

    lax.fori_loop(0, _CC, _orow, 0)

    def _zcp(k, carry):
        off = sid * rpt + k * _STG
        pltpu.sync_copy(stg16, cntsh.at[pl.ds(off, _STG)])
        return carry

    lax.fori_loop(0, nstg, _zcp, 0)

    plsc.subcore_barrier()

    ebase = cid * epc + sid * epw

    def _chunk(ci, carry):
        eb = ebase + ci * _CC
        pltpu.sync_copy(dst_hbm.at[pl.ds(eb, _CC)], dsti)
        pltpu.sync_copy(onesv, cntsh.at[dsti], add=True)
        return carry

    lax.fori_loop(0, nchunk, _chunk, 0)

    plsc.subcore_barrier()

    def _rd(k, carry):
        off = sid * rpt + k * _STG
        pltpu.sync_copy(cntsh.at[pl.ds(off, _STG)], stg16)
        pltpu.sync_copy(stg16, cnt_out.at[cid, pl.ds(off, _STG)])
        return carry

    lax.fori_loop(0, nstg, _rd, 0)


def _sc_count_pass(dst, npad):
    mesh = plsc.VectorSubcoreMesh(core_axis_name="c", subcore_axis_name="s")
    kern = pl.kernel(
        _sc_count_body,
        mesh=mesh,
        out_type=jax.ShapeDtypeStruct((2, npad, 128), jnp.float32),
        scratch_types=[
            pltpu.VMEM((_CC,), jnp.int32),
            pltpu.VMEM((_CC, 128), jnp.float32),
            pltpu.VMEM((_STG, 128), jnp.float32),
            pltpu.VMEM_SHARED((npad, 128), jnp.float32),
        ],
    )
    return kern(dst)


def _final_body(x_ref, aggp_ref, cntp_ref, g_ref, b_ref, out_ref):
    n = x_ref.shape[0]
    agg = (aggp_ref[0] + aggp_ref[1])[:n]
    cnt = (cntp_ref[0] + cntp_ref[1])[:n]
    c = cnt[:, 0:1]
    out = x_ref[...] + agg / jnp.maximum(c, 1.0)
    m = jnp.mean(out, axis=0, keepdims=True)
    d = out - m
    v = jnp.mean(d * d, axis=0, keepdims=True)
    out_ref[...] = d * lax.rsqrt(v + 1e-5) * g_ref[...] + b_ref[...]


def _final(x, aggp, cntp, gamma, beta):
    n, f = x.shape
    return pl.pallas_call(
        _final_body,
        out_shape=jax.ShapeDtypeStruct((n, f), jnp.float32),
    )(x, aggp, cntp, gamma, beta)


def kernel(x, edge_index, edge_attr, Wf, bf, Ws, bs, gamma, beta):
    n, f = x.shape
    src = edge_index[0]
    dst = edge_index[1]
    wd = jnp.concatenate([Wf[:f], Ws[:f]], axis=1)
    wsrc = jnp.concatenate([Wf[f:2 * f], Ws[f:2 * f]], axis=1)
    we = jnp.concatenate([Wf[2 * f:], Ws[2 * f:]], axis=1)
    bias = jnp.concatenate([bf, bs]).reshape(1, 2 * f)
    npad = ((n + 16 * _STG - 1) // (16 * _STG)) * (16 * _STG)
    pd, psrc = _nodeproj(x, wd, wsrc, bias)
    ep = _edgeproj(edge_attr, we)
    cntp = _sc_count_pass(dst, npad)
    aggp = _sc_edge_pass(dst, src, pd, psrc, ep, npad)
    return _final(
        x, aggp, cntp, gamma.reshape(1, f), beta.reshape(1, f)
    )


# speedup vs baseline: 2.4411x; 2.4411x over previous
import functools

import jax
import jax.numpy as jnp
from jax import lax
from jax.experimental import pallas as pl
from jax.experimental.pallas import tpu as pltpu
from jax.experimental.pallas import tpu_sc as plsc


def _nodeproj_body(x_ref, wd_ref, ws_ref, b_ref, pd_ref, ps_ref):
    xv = x_ref[...]
    pd_ref[...] = (
        jnp.dot(xv, wd_ref[...], preferred_element_type=jnp.float32) + b_ref[...]
    )
    ps_ref[...] = jnp.dot(xv, ws_ref[...], preferred_element_type=jnp.float32)


def _nodeproj(x, wd, ws, bias):
    n, f = x.shape
    f2 = wd.shape[1]
    bn = 2000
    return pl.pallas_call(
        _nodeproj_body,
        grid=(n // bn,),
        in_specs=[
            pl.BlockSpec((bn, f), lambda i: (i, 0)),
            pl.BlockSpec((f, f2), lambda i: (0, 0)),
            pl.BlockSpec((f, f2), lambda i: (0, 0)),
            pl.BlockSpec((1, f2), lambda i: (0, 0)),
        ],
        out_specs=[
            pl.BlockSpec((bn, f2), lambda i: (i, 0)),
            pl.BlockSpec((bn, f2), lambda i: (i, 0)),
        ],
        out_shape=[jax.ShapeDtypeStruct((n, f2), jnp.float32)] * 2,
    )(x, wd, ws, bias)


def _edgeproj_body(ea_ref, we_ref, ep_ref):
    ep_ref[...] = jnp.dot(
        ea_ref[...], we_ref[...], preferred_element_type=jnp.float32
    )


def _edgeproj(edge_attr, we):
    e, d = edge_attr.shape
    f2 = we.shape[1]
    be = 3200
    return pl.pallas_call(
        _edgeproj_body,
        grid=(e // be,),
        in_specs=[
            pl.BlockSpec((be, d), lambda i: (i, 0)),
            pl.BlockSpec((d, f2), lambda i: (0, 0)),
        ],
        out_specs=pl.BlockSpec((be, f2), lambda i: (i, 0)),
        out_shape=jax.ShapeDtypeStruct((e, f2), jnp.float32),
    )(edge_attr, we)


_C = 40
_STG = 32


def _compute_chunk(pdv, psv, epv, msgv):
    @plsc.parallel_loop(0, _C, unroll=2)
    def _row(r):
        for j in range(8):
            gsl = pl.ds(16 * j, 16)
            csl = pl.ds(128 + 16 * j, 16)
            g = pdv[r, gsl] + psv[r, gsl] + epv[r, gsl]
            cz = pdv[r, csl] + psv[r, csl] + epv[r, csl]
            eg = jnp.exp(-g)
            u = jnp.exp(-jnp.abs(cz))
            u2 = u + 2.0
            t = u / u2
            t2 = t * t
            poly = 1.0 + t2 * (0.33333334 + t2 * (0.2 + t2 * 0.14285715))
            num = jnp.maximum(cz, 0.0) * u2 + 2.0 * u * poly
            den = u2 * (1.0 + eg)
            msgv[r, gsl] = num / den


def _sc_body(
    dst_hbm, src_hbm, pd_hbm, ps_hbm, ep_hbm,
    agg_out,
    dsti, srci, dsts, pdv, psv, epv, msgv, stg, aggsh,
    sem0, sem1, sem2, sem3,
):
    npad = aggsh.shape[0]
    e = dst_hbm.shape[0]
    cid = lax.axis_index("c")
    sid = lax.axis_index("s")
    epc = e // 2
    epw = epc // 16
    nchunk = epw // _C
    rpt = npad // 16
    nstg = rpt // _STG

    zero16 = jnp.zeros((16,), jnp.float32)

    def _zrow(r, carry):
        for j in range(8):
            stg[r, pl.ds(16 * j, 16)] = zero16
        return carry

    lax.fori_loop(0, _STG, _zrow, 0)

    def _zcp(k, carry):
        off = sid * rpt + k * _STG
        pltpu.sync_copy(stg, aggsh.at[pl.ds(off, _STG)])
        return carry

    lax.fori_loop(0, nstg, _zcp, 0)

    plsc.subcore_barrier()

    ebase = cid * epc + sid * epw

    def _gathers(eb):
        cp0 = pltpu.async_copy(pd_hbm.at[dsti], pdv, sem0)
        cp1 = pltpu.async_copy(ps_hbm.at[srci], psv, sem1)
        cp2 = pltpu.async_copy(ep_hbm.at[pl.ds(eb, _C)], epv, sem2)
        return cp0, cp1, cp2

    def _load_idx(eb):
        pltpu.sync_copy(dst_hbm.at[pl.ds(eb, _C)], dsti)
        pltpu.sync_copy(src_hbm.at[pl.ds(eb, _C)], srci)

    _load_idx(ebase)
    _gathers(ebase)

    def _wait_gathers():
        pltpu.make_async_copy(pd_hbm.at[dsti], pdv, sem0).wait()
        pltpu.make_async_copy(ps_hbm.at[srci], psv, sem1).wait()
        pltpu.make_async_copy(ep_hbm.at[pl.ds(0, _C)], epv, sem2).wait()

    def _chunk(ci, carry):
        _wait_gathers()

        _compute_chunk(pdv, psv, epv, msgv)

        for o in (0, 16, 24):
            dsts[pl.ds(o, 16)] = dsti[pl.ds(o, 16)]
        sc = pltpu.async_copy(msgv, aggsh.at[dsts], sem3, add=True)

        eb = ebase + jnp.minimum(ci + 1, nchunk - 1) * _C
        _load_idx(eb)
        _gathers(eb)
        sc.wait()
        return carry

    lax.fori_loop(0, nchunk, _chunk, 0)

    _wait_gathers()

    plsc.subcore_barrier()

    def _rd(k, carry):
        off = sid * rpt + k * _STG
        pltpu.sync_copy(aggsh.at[pl.ds(off, _STG)], stg)
        pltpu.sync_copy(stg, agg_out.at[cid, pl.ds(off, _STG)])
        return carry

    lax.fori_loop(0, nstg, _rd, 0)


def _sc_edge_pass(dst, src, pd, ps, ep, npad):
    mesh = plsc.VectorSubcoreMesh(core_axis_name="c", subcore_axis_name="s")
    kern = pl.kernel(
        _sc_body,
        mesh=mesh,
        out_type=jax.ShapeDtypeStruct((2, npad, 128), jnp.float32),
        scratch_types=[
            pltpu.VMEM((_C,), jnp.int32),
            pltpu.VMEM((_C,), jnp.int32),
            pltpu.VMEM((_C,), jnp.int32),
            pltpu.VMEM((_C, 256), jnp.float32),
            pltpu.VMEM((_C, 256), jnp.float32),
            pltpu.VMEM((_C, 256), jnp.float32),
            pltpu.VMEM((_C, 128), jnp.float32),
            pltpu.VMEM((_STG, 128), jnp.float32),
            pltpu.VMEM_SHARED((npad, 128), jnp.float32),
            pltpu.SemaphoreType.DMA,
            pltpu.SemaphoreType.DMA,
            pltpu.SemaphoreType.DMA,
            pltpu.SemaphoreType.DMA,
        ],
    )
    return kern(dst, src, pd, ps, ep)


_CC = 80


def _sc_count_body(dst_hbm, cnt_out, dsti, onesv, stg16, cntsh):
    npad = cntsh.shape[0]
    e = dst_hbm.shape[0]
    cid = lax.axis_index("c")
    sid = lax.axis_index("s")
    epc = e // 2
    epw = epc // 16
    nchunk = epw // _CC
    rpt = npad // 16
    nstg = rpt // _STG

    zero16 = jnp.zeros((16,), jnp.float32)
    lanes = lax.iota(jnp.int32, 16)
    onerow = jnp.where(lanes == 0, 1.0, 0.0).astype(jnp.float32)

    def _init(r, carry):
        for j in range(8):
            stg16[r, pl.ds(16 * j, 16)] = zero16
        return carry

    lax.fori_loop(0, _STG, _init, 0)

    def _orow(r, carry):
        onesv[r, pl.ds(0, 16)] = onerow
        for j in range(1, 8):
            onesv[r, pl.ds(16 * j, 16)] = zero16
        return carry

    lax.fori_loop(0, _CC, _orow, 0)

    def _zcp(k, carry):
        off = sid * rpt + k * _STG
        pltpu.sync_copy(stg16, cntsh.at[pl.ds(off, _STG)])
        return carry

    lax.fori_loop(0, nstg, _zcp, 0)

    plsc.subcore_barrier()

    ebase = cid * epc + sid * epw

    def _chunk(ci, carry):
        eb = ebase + ci * _CC
        pltpu.sync_copy(dst_hbm.at[pl.ds(eb, _CC)], dsti)
        pltpu.sync_copy(onesv, cntsh.at[dsti], add=True)
        return carry

    lax.fori_loop(0, nchunk, _chunk, 0)

    plsc.subcore_barrier()

    def _rd(k, carry):
        off = sid * rpt + k * _STG
        pltpu.sync_copy(cntsh.at[pl.ds(off, _STG)], stg16)
        pltpu.sync_copy(stg16, cnt_out.at[cid, pl.ds(off, _STG)])
        return carry

    lax.fori_loop(0, nstg, _rd, 0)


def _sc_count_pass(dst, npad):
    mesh = plsc.VectorSubcoreMesh(core_axis_name="c", subcore_axis_name="s")
    kern = pl.kernel(
        _sc_count_body,
        mesh=mesh,
        out_type=jax.ShapeDtypeStruct((2, npad, 128), jnp.float32),
        scratch_types=[
            pltpu.VMEM((_CC,), jnp.int32),
            pltpu.VMEM((_CC, 128), jnp.float32),
            pltpu.VMEM((_STG, 128), jnp.float32),
            pltpu.VMEM_SHARED((npad, 128), jnp.float32),
        ],
    )
    return kern(dst)


def _final_body(x_ref, aggp_ref, cntp_ref, g_ref, b_ref, out_ref):
    n = x_ref.shape[0]
    agg = (aggp_ref[0] + aggp_ref[1])[:n]
    cnt = (cntp_ref[0] + cntp_ref[1])[:n]
    c = cnt[:, 0:1]
    out = x_ref[...] + agg / jnp.maximum(c, 1.0)
    m = jnp.mean(out, axis=0, keepdims=True)
    d = out - m
    v = jnp.mean(d * d, axis=0, keepdims=True)
    out_ref[...] = d * lax.rsqrt(v + 1e-5) * g_ref[...] + b_ref[...]


def _final(x, aggp, cntp, gamma, beta):
    n, f = x.shape
    return pl.pallas_call(
        _final_body,
        out_shape=jax.ShapeDtypeStruct((n, f), jnp.float32),
    )(x, aggp, cntp, gamma, beta)


def kernel(x, edge_index, edge_attr, Wf, bf, Ws, bs, gamma, beta):
    n, f = x.shape
    src = edge_index[0]
    dst = edge_index[1]
    wd = jnp.concatenate([Wf[:f], Ws[:f]], axis=1)
    wsrc = jnp.concatenate([Wf[f:2 * f], Ws[f:2 * f]], axis=1)
    we = jnp.concatenate([Wf[2 * f:], Ws[2 * f:]], axis=1)
    bias = jnp.concatenate([bf, bs]).reshape(1, 2 * f)
    npad = ((n + 16 * _STG - 1) // (16 * _STG)) * (16 * _STG)
    pd, psrc = _nodeproj(x, wd, wsrc, bias)
    ep = _edgeproj(edge_attr, we)
    cntp = _sc_count_pass(dst, npad)
    aggp = _sc_edge_pass(dst, src, pd, psrc, ep, npad)
    return _final(
        x, aggp, cntp, gamma.reshape(1, f), beta.reshape(1, f)
    )
